# baseline (speedup 1.0000x reference)
import jax
import jax.numpy as jnp
from jax import lax
from jax.experimental import pallas as pl
from jax.experimental.pallas import tpu as pltpu
from jax.experimental.pallas import tpu_sc as plsc

D = 1500
DP2 = 2048
W_IN = 2 * D
W_OUT = 2 * DP2
N = 16384

NUM_WORKERS = 32
LANES = 16
CW = 2048
NCOL = N // CW
G_DATA = 188
G_HOLE = 70
H_PAD = G_HOLE * LANES
MAX_IT = 48


def _sc_body(xt_hbm, s_hbm, h_hbm, outT_hbm,
             s_v, h_v, ib0, ib1, zb, si0, si1, so0, so1, sh):
    w = lax.axis_index("s") * 2 + lax.axis_index("c")
    ucnt = jnp.where(w < 28, 6, 5)
    ustart = 6 * w - jnp.maximum(w - 28, 0)
    nit = 8 * ucnt

    pltpu.sync_copy(s_hbm, s_v)
    pltpu.sync_copy(h_hbm, h_v)

    ibufs, sis, sos = (ib0, ib1), (si0, si1), (so0, so1)

    def c0_of(it):
        u = ustart + (it // 8)
        return jnp.minimum(16 * u, W_IN - 16)

    def col_of(it):
        return (it % 8) * CW

    def start_in(it, p):
        pltpu.async_copy(
            xt_hbm.at[pl.ds(c0_of(it), LANES), pl.ds(col_of(it), CW)],
            ibufs[p], sis[p])

    start_in(0, 0)
    start_in(1, 1)

    zv = jnp.zeros((LANES,), jnp.float32)

    @plsc.parallel_loop(0, CW // LANES, unroll=4)
    def _zero(j):
        for r in range(LANES):
            zb[r, pl.ds(j * LANES, LANES)] = zv

    def phase(it, p):
        @pl.when(it < 24)
        def _hole():
            hc = w + 32 * (it // 8)

            @pl.when(hc < G_HOLE)
            def _():
                hv = h_v[pl.ds(hc * LANES, LANES)]
                pltpu.async_copy(
                    zb, outT_hbm.at[hv, pl.ds(col_of(it), CW)], sh)

        @pl.when(it < nit)
        def _data():
            pltpu.make_async_copy(
                xt_hbm.at[pl.ds(0, LANES), pl.ds(0, CW)],
                ibufs[p], sis[p]).wait()
            sv = s_v[pl.ds(c0_of(it), LANES)]
            dst = outT_hbm.at[sv, pl.ds(col_of(it), CW)]
            pltpu.async_copy(ibufs[p], dst, sos[p])
            pltpu.make_async_copy(ibufs[p], dst, sos[p]).wait()

            @pl.when(it + 2 < nit)
            def _():
                start_in(it + 2, p)

    def pair(it2, c):
        phase(2 * it2, 0)
        phase(2 * it2 + 1, 1)
        return c

    lax.fori_loop(0, MAX_IT // 2, pair, 0)

    hv0 = h_v[pl.ds(0, LANES)]
    hole_dst = outT_hbm.at[hv0, pl.ds(0, CW)]
    for k in range(16):
        pltpu.make_async_copy(zb, hole_dst, sh).wait()

    @pl.when(w < 6)
    def _drain_rest():
        for k in range(8):
            pltpu.make_async_copy(zb, hole_dst, sh).wait()


@jax.jit
def kernel(X, idx):
    XT = jnp.swapaxes(X, 0, 1)
    s = jnp.concatenate([idx, idx + DP2])
    inv = jnp.full((W_OUT,), -1, jnp.int32).at[s].set(
        jnp.arange(W_IN, dtype=jnp.int32))
    holes = jnp.nonzero(inv < 0, size=H_PAD,
                        fill_value=W_OUT)[0].astype(jnp.int32)
    holes = jnp.where(jnp.arange(H_PAD) < W_OUT - W_IN, holes,
                      holes[W_OUT - W_IN - 1])

    run = pl.kernel(
        _sc_body,
        out_type=jax.ShapeDtypeStruct((W_OUT, N), jnp.float32),
        mesh=plsc.VectorSubcoreMesh(core_axis_name="c", subcore_axis_name="s"),
        compiler_params=pltpu.CompilerParams(needs_layout_passes=False),
        scratch_types=[
            pltpu.VMEM((W_IN,), jnp.int32),
            pltpu.VMEM((H_PAD,), jnp.int32),
            pltpu.VMEM((LANES, CW), jnp.float32),
            pltpu.VMEM((LANES, CW), jnp.float32),
            pltpu.VMEM((LANES, CW), jnp.float32),
            pltpu.SemaphoreType.DMA,
            pltpu.SemaphoreType.DMA,
            pltpu.SemaphoreType.DMA,
            pltpu.SemaphoreType.DMA,
            pltpu.SemaphoreType.DMA,
        ],
    )
    outT = run(XT, s, holes)
    return jnp.swapaxes(outT, 0, 1)


# speedup vs baseline: 1.1543x; 1.1543x over previous
import jax
import jax.numpy as jnp
from jax import lax
from jax.experimental import pallas as pl
from jax.experimental.pallas import tpu as pltpu
from jax.experimental.pallas import tpu_sc as plsc

D = 1500
DP2 = 2048
W_IN = 2 * D
W_OUT = 2 * DP2
N = 16384

NUM_WORKERS = 32
ROWS_PER_W = N // NUM_WORKERS
BLK = 8
NBLK = ROWS_PER_W // BLK
LANES = 16
FULLSTEPS = W_IN // LANES
TAIL = W_IN - FULLSTEPS * LANES


def _sc_body(x_hbm, s_hbm, out_hbm,
             s_v, in0, in1, ob0, ob1, si0, si1, so0, so1):
    wid = lax.axis_index("s") * 2 + lax.axis_index("c")
    base = wid * ROWS_PER_W
    ins, obs, sis, sos = (in0, in1), (ob0, ob1), (si0, si1), (so0, so1)

    pltpu.sync_copy(s_hbm, s_v)

    pltpu.async_copy(x_hbm.at[pl.ds(base, BLK)], in0, si0)
    pltpu.async_copy(x_hbm.at[pl.ds(base + BLK, BLK)], in1, si1)

    zv = jnp.zeros((LANES,), jnp.float32)

    @plsc.parallel_loop(0, W_OUT // LANES, unroll=4)
    def _zero(j):
        for r in range(BLK):
            ob0[r, pl.ds(j * LANES, LANES)] = zv
            ob1[r, pl.ds(j * LANES, LANES)] = zv

    rvs = [jnp.full((LANES,), r, jnp.int32) for r in range(BLK)]
    tailmask = lax.iota(jnp.int32, LANES) >= (LANES - TAIL)

    def compute(inb, ob):
        @plsc.parallel_loop(0, FULLSTEPS, unroll=4)
        def _scatter(j):
            sv = s_v[pl.ds(j * LANES, LANES)]
            for r in range(BLK):
                vals = inb[r, pl.ds(j * LANES, LANES)]
                plsc.store_scatter(ob, [rvs[r], sv], vals)
        sv = s_v[pl.ds(W_IN - LANES, LANES)]
        for r in range(BLK):
            vals = inb[r, pl.ds(W_IN - LANES, LANES)]
            plsc.store_scatter(ob, [rvs[r], sv], vals, mask=tailmask)

    def pair(t, c):
        for p in range(2):
            b = t * 2 + p
            rowbase = base + b * BLK
            pltpu.make_async_copy(x_hbm.at[pl.ds(0, BLK)], ins[p],
                                  sis[p]).wait()

            @pl.when(t > 0)
            def _wait_out():
                pltpu.make_async_copy(obs[p], out_hbm.at[pl.ds(0, BLK)],
                                      sos[p]).wait()

            compute(ins[p], obs[p])
            pltpu.async_copy(obs[p], out_hbm.at[pl.ds(rowbase, BLK)], sos[p])

            @pl.when(b + 2 < NBLK)
            def _next_in():
                pltpu.async_copy(x_hbm.at[pl.ds(rowbase + 2 * BLK, BLK)],
                                 ins[p], sis[p])
        return c

    lax.fori_loop(0, NBLK // 2, pair, 0)
    pltpu.make_async_copy(ob0, out_hbm.at[pl.ds(0, BLK)], so0).wait()
    pltpu.make_async_copy(ob1, out_hbm.at[pl.ds(0, BLK)], so1).wait()


@jax.jit
def kernel(X, idx):
    s = jnp.concatenate([idx, idx + DP2])
    run = pl.kernel(
        _sc_body,
        out_type=jax.ShapeDtypeStruct((N, W_OUT), jnp.float32),
        mesh=plsc.VectorSubcoreMesh(core_axis_name="c", subcore_axis_name="s"),
        compiler_params=pltpu.CompilerParams(needs_layout_passes=False),
        scratch_types=[
            pltpu.VMEM((W_IN,), jnp.int32),
            pltpu.VMEM((BLK, W_IN), jnp.float32),
            pltpu.VMEM((BLK, W_IN), jnp.float32),
            pltpu.VMEM((BLK, W_OUT), jnp.float32),
            pltpu.VMEM((BLK, W_OUT), jnp.float32),
            pltpu.SemaphoreType.DMA,
            pltpu.SemaphoreType.DMA,
            pltpu.SemaphoreType.DMA,
            pltpu.SemaphoreType.DMA,
        ],
    )
    return run(X, s)
